# baseline (speedup 1.0000x reference)
import functools

import jax
import jax.numpy as jnp
from jax import lax
from jax.experimental import pallas as pl
from jax.experimental.pallas import tpu as pltpu
from jax.experimental.pallas import tpu_sc as plsc

Q = 1024
D = 128
N = 100000
W = 32
NCHUNK = N // W
KB = 512
NKB = 196
NPAD = NKB * KB
CPB = KB // W
NCHUNK_PAD = NKB * CPB
C = 128
CAND = C * W
K_STATIC = 100
RB = 32
RBM = 32
I32_MIN = -(2**31)


def _bitonic_desc(v, idx, n):
    io = lax.broadcasted_iota(jnp.int32, v.shape, 1)
    k_ = 2
    while k_ <= n:
        j = k_ // 2
        while j >= 1:
            m_lower = (io & j) == 0
            pv = jnp.where(m_lower, jnp.roll(v, -j, axis=1), jnp.roll(v, j, axis=1))
            pi = jnp.where(m_lower, jnp.roll(idx, -j, axis=1), jnp.roll(idx, j, axis=1))
            m_dir = (io & k_) == 0
            want_max = m_lower == m_dir
            gt = (v > pv) | ((v == pv) & (idx < pi))
            take = want_max ^ gt
            v = jnp.where(take, pv, v)
            idx = jnp.where(take, pi, idx)
            j //= 2
        k_ *= 2
    return v, idx


def _mm_chunkmax_kernel(q_ref, k_ref, s_ref, cm_ref):
    i = pl.program_id(0)
    s = lax.dot_general(q_ref[...], k_ref[...], (((1,), (1,)), ((), ())),
                        preferred_element_type=jnp.float32)
    s_ref[...] = s
    st = lax.dot_general(k_ref[...], q_ref[...], (((1,), (1,)), ((), ())),
                         preferred_element_type=jnp.float32)
    cm = jnp.max(st.reshape(CPB, W, Q), axis=1)
    chunk_id = i * CPB + lax.broadcasted_iota(jnp.int32, (CPB, Q), 0)
    cm = jnp.where(chunk_id < NCHUNK, cm, -jnp.inf)
    cm_ref[0] = cm


def _to_su(v):
    b = lax.bitcast_convert_type(v, jnp.int32)
    return jnp.where(b >= 0, b, b ^ jnp.int32(0x7FFFFFFF))


def _radix_thresholds(su, ids, count, id_bits):
    r = su.shape[0]
    p = jnp.full((r, 1), I32_MIN, jnp.int32)
    for b in range(31, -1, -1):
        inc = jnp.int32(I32_MIN if b == 31 else 1 << b)
        cand = p + inc
        cnt = jnp.sum((su >= cand).astype(jnp.int32), axis=1, keepdims=True)
        p = jnp.where(cnt >= count, cand, p)
    tv = p
    c_gt = jnp.sum((su > tv).astype(jnp.int32), axis=1, keepdims=True)
    tie = su == tv
    n_tie = jnp.sum(tie.astype(jnp.int32), axis=1, keepdims=True)
    t_sel = n_tie - (count - c_gt) + 1
    q = jnp.zeros((r, 1), jnp.int32)
    for b in range(id_bits - 1, -1, -1):
        cand = q + jnp.int32(1 << b)
        cnt = jnp.sum((tie & (ids >= cand)).astype(jnp.int32), axis=1,
                      keepdims=True)
        q = jnp.where(cnt >= t_sel, cand, q)
    return tv, q


def _pack_thr(tv, tid, shape):
    lane = lax.broadcasted_iota(jnp.int32, shape, 1)
    return jnp.where(lane == 0, tv, jnp.where(lane == 1, tid, 0))


def _chunk_thr_kernel(cm_ref, thr_ref):
    su = _to_su(cm_ref[...])
    ids = lax.broadcasted_iota(jnp.int32, (RB, NCHUNK_PAD), 1)
    tv, tid = _radix_thresholds(su, ids, C, 12)
    thr_ref[...] = _pack_thr(tv, tid, (RB, C))


def _cand_thr_kernel(cand_ref, cid_ref, thr_ref, gid_ref):
    b = pl.program_id(0)
    v = cand_ref[...]
    cflat = cid_ref[...]
    rows = b * RB + lax.broadcasted_iota(jnp.int32, (RB, C), 0)
    c = cflat - rows * NCHUNK_PAD
    rep = jnp.repeat(c, W, axis=1)
    off = lax.broadcasted_iota(jnp.int32, (RB, CAND), 1) % W
    gid = rep * W + off
    su = _to_su(v)
    tv, tid = _radix_thresholds(su, gid, K_STATIC, 17)
    thr_ref[...] = _pack_thr(tv, tid, (RB, C))
    gid_ref[...] = gid


def _mini_sort_kernel(v_ref, i_ref, vo_ref, io_ref):
    v = v_ref[...]
    idx = i_ref[...]
    lane = lax.broadcasted_iota(jnp.int32, (RBM, C), 1)
    live = lane < K_STATIC
    v = jnp.where(live, v, -jnp.inf)
    idx = jnp.where(live, idx, jnp.int32(2**30) + lane)
    v, idx = _bitonic_desc(v, idx, C)
    vo_ref[...] = v
    io_ref[...] = idx


def _bcast16(vec16, lane):
    idx = jnp.full((16, 1), lane, jnp.int32)
    return lax.gather(
        vec16, idx,
        dimension_numbers=lax.GatherDimensionNumbers(
            offset_dims=(), collapsed_slice_dims=(0,), start_index_map=(0,)),
        slice_sizes=(1,), mode=lax.GatherScatterMode.PROMISE_IN_BOUNDS)


_MESH = None


def _sc_mesh():
    global _MESH
    if _MESH is None:
        _MESH = plsc.VectorSubcoreMesh(core_axis_name="c", subcore_axis_name="s")
    return _MESH


def _sc_info():
    info = plsc.get_sparse_core_info()
    return info.num_cores, info.num_subcores


def _make_sc_gather(d, B, chunk, sc_tiling):
    nc, ns = _sc_info()
    nw = nc * ns
    b_per_w = B // nw
    n_iter = b_per_w // chunk

    @functools.partial(
        pl.kernel, mesh=_sc_mesh(),
        out_type=jax.ShapeDtypeStruct((B, d), jnp.float32),
        scratch_types=[
            pltpu.VMEM((chunk,), jnp.int32),
            pltpu.VMEM((chunk, d), jnp.float32),
            pltpu.SemaphoreType.DMA,
        ],
        compiler_params=pltpu.CompilerParams(use_tc_tiling_on_sc=not sc_tiling),
    )
    def k(table_hbm, idx_hbm, out_hbm, idx_v, rows_v, sem):
        wid = lax.axis_index("s") * nc + lax.axis_index("c")
        base = wid * b_per_w
        for h in range(n_iter):
            pltpu.sync_copy(idx_hbm.at[pl.ds(base + h * chunk, chunk)], idx_v)
            pltpu.async_copy(table_hbm.at[idx_v], rows_v, sem).wait()
            pltpu.sync_copy(rows_v, out_hbm.at[pl.ds(base + h * chunk, chunk)])

    return k


def _make_sc_compact_chunks():
    nc, ns = _sc_info()
    rows_per_w = Q // (nc * ns)

    @functools.partial(
        pl.kernel, mesh=_sc_mesh(),
        out_type=jax.ShapeDtypeStruct((Q * C,), jnp.int32),
        scratch_types=[
            pltpu.VMEM((NCHUNK_PAD,), jnp.float32),
            pltpu.VMEM((16,), jnp.int32),
            pltpu.VMEM((C + 16,), jnp.int32),
        ],
        compiler_params=pltpu.CompilerParams(needs_layout_passes=False),
    )
    def k(cm_hbm, thr_hbm, out_hbm, cmv, thrv, outv):
        wid = lax.axis_index("s") * nc + lax.axis_index("c")

        @pl.loop(0, rows_per_w)
        def row_body(i):
            r = wid * rows_per_w + i
            pltpu.sync_copy(cm_hbm.at[pl.ds(r * NCHUNK_PAD, NCHUNK_PAD)], cmv)
            pltpu.sync_copy(thr_hbm.at[pl.ds(r * C, 16)], thrv)
            thrvec = thrv[...]
            tv = _bcast16(thrvec, 0)
            tid = _bcast16(thrvec, 1)
            fb = r * NCHUNK_PAD

            @pl.loop(0, NCHUNK_PAD // 16,
                     init_carry=jnp.zeros((16,), jnp.int32), unroll=8)
            def vreg_body(t, off):
                x = cmv[pl.ds(t * 16, 16)]
                bb = lax.bitcast_convert_type(x, jnp.int32)
                su = jnp.where(bb >= 0, bb, bb ^ jnp.int32(0x7FFFFFFF))
                ids = t * 16 + lax.iota(jnp.int32, 16)
                msk = (su > tv) | ((su == tv) & (ids <= tid))
                cs = plsc.cumsum(msk.astype(jnp.int32))
                pos = jnp.where(msk, off + cs - 1, C + 8)
                plsc.store_scatter(outv, [pos], fb + ids)
                return off + _bcast16(cs, 15)

            pltpu.sync_copy(outv.at[pl.ds(0, C)], out_hbm.at[pl.ds(r * C, C)])

    return k


def _make_sc_compact_cands():
    nc, ns = _sc_info()
    rows_per_w = Q // (nc * ns)

    @functools.partial(
        pl.kernel, mesh=_sc_mesh(),
        out_type=(jax.ShapeDtypeStruct((Q * C,), jnp.float32),
                  jax.ShapeDtypeStruct((Q * C,), jnp.int32)),
        scratch_types=[
            pltpu.VMEM((CAND,), jnp.float32),
            pltpu.VMEM((CAND,), jnp.int32),
            pltpu.VMEM((16,), jnp.int32),
            pltpu.VMEM((C + 16,), jnp.float32),
            pltpu.VMEM((C + 16,), jnp.int32),
        ],
        compiler_params=pltpu.CompilerParams(needs_layout_passes=False),
    )
    def k(vals_hbm, gid_hbm, thr_hbm, outv_hbm, outi_hbm, vv, gv, thrv,
          outv, outi):
        wid = lax.axis_index("s") * nc + lax.axis_index("c")
        neginf = jnp.full((16,), -jnp.inf, jnp.float32)
        bigid = jnp.full((16,), 2**30, jnp.int32)

        @pl.loop(0, rows_per_w)
        def row_body(i):
            r = wid * rows_per_w + i
            pltpu.sync_copy(vals_hbm.at[pl.ds(r * CAND, CAND)], vv)
            pltpu.sync_copy(gid_hbm.at[pl.ds(r * CAND, CAND)], gv)
            pltpu.sync_copy(thr_hbm.at[pl.ds(r * C, 16)], thrv)
            thrvec = thrv[...]
            tv = _bcast16(thrvec, 0)
            tid = _bcast16(thrvec, 1)
            for base in (96, 112, 128):
                outv[pl.ds(base, 16)] = neginf
                outi[pl.ds(base, 16)] = bigid

            @pl.loop(0, CAND // 16, init_carry=jnp.zeros((16,), jnp.int32),
                     unroll=8)
            def vreg_body(t, off):
                x = vv[pl.ds(t * 16, 16)]
                g = gv[pl.ds(t * 16, 16)]
                bb = lax.bitcast_convert_type(x, jnp.int32)
                su = jnp.where(bb >= 0, bb, bb ^ jnp.int32(0x7FFFFFFF))
                msk = (su > tv) | ((su == tv) & (g <= tid))
                cs = plsc.cumsum(msk.astype(jnp.int32))
                pos = jnp.where(msk, off + cs - 1, C + 8)
                plsc.store_scatter(outv, [pos], x)
                plsc.store_scatter(outi, [pos], g)
                return off + _bcast16(cs, 15)

            pltpu.sync_copy(outv.at[pl.ds(0, C)],
                            outv_hbm.at[pl.ds(r * C, C)])
            pltpu.sync_copy(outi.at[pl.ds(0, C)],
                            outi_hbm.at[pl.ds(r * C, C)])

    return k


def kernel(queries, index, k):
    index_p = jnp.pad(index, ((0, NPAD - N), (0, 0)))
    s, cm3 = pl.pallas_call(
        _mm_chunkmax_kernel,
        grid=(NKB,),
        in_specs=[pl.BlockSpec((Q, D), lambda i: (0, 0)),
                  pl.BlockSpec((KB, D), lambda i: (i, 0))],
        out_specs=[pl.BlockSpec((Q, KB), lambda i: (0, i)),
                   pl.BlockSpec((1, CPB, Q), lambda i: (i, 0, 0))],
        out_shape=[jax.ShapeDtypeStruct((Q, NPAD), jnp.float32),
                   jax.ShapeDtypeStruct((NKB, CPB, Q), jnp.float32)],
    )(queries, index_p)
    cm = cm3.reshape(NCHUNK_PAD, Q).T
    thr = pl.pallas_call(
        _chunk_thr_kernel,
        grid=(Q // RB,),
        in_specs=[pl.BlockSpec((RB, NCHUNK_PAD), lambda b: (b, 0))],
        out_specs=pl.BlockSpec((RB, C), lambda b: (b, 0)),
        out_shape=jax.ShapeDtypeStruct((Q, C), jnp.int32),
    )(cm)
    chunk_flat = _make_sc_compact_chunks()(cm.reshape(-1), thr.reshape(-1))
    table = s.reshape(Q * NCHUNK_PAD, W)
    cand = _make_sc_gather(W, Q * C, 2048, True)(table, chunk_flat)
    cand = cand.reshape(Q, CAND)
    chunk_flat = chunk_flat.reshape(Q, C)
    thr2, gids = pl.pallas_call(
        _cand_thr_kernel,
        grid=(Q // RB,),
        in_specs=[pl.BlockSpec((RB, CAND), lambda b: (b, 0)),
                  pl.BlockSpec((RB, C), lambda b: (b, 0))],
        out_specs=[pl.BlockSpec((RB, C), lambda b: (b, 0)),
                   pl.BlockSpec((RB, CAND), lambda b: (b, 0))],
        out_shape=[jax.ShapeDtypeStruct((Q, C), jnp.int32),
                   jax.ShapeDtypeStruct((Q, CAND), jnp.int32)],
    )(cand, chunk_flat)
    topv, topi = _make_sc_compact_cands()(
        cand.reshape(-1), gids.reshape(-1), thr2.reshape(-1))
    topv = topv.reshape(Q, C)
    topi = topi.reshape(Q, C)
    vals, ids = pl.pallas_call(
        _mini_sort_kernel,
        grid=(Q // RBM,),
        in_specs=[pl.BlockSpec((RBM, C), lambda b: (b, 0)),
                  pl.BlockSpec((RBM, C), lambda b: (b, 0))],
        out_specs=[pl.BlockSpec((RBM, C), lambda b: (b, 0)),
                   pl.BlockSpec((RBM, C), lambda b: (b, 0))],
        out_shape=[jax.ShapeDtypeStruct((Q, C), jnp.float32),
                   jax.ShapeDtypeStruct((Q, C), jnp.int32)],
    )(topv, topi)
    cols = jnp.minimum(jnp.arange(K_STATIC), k - 1)
    top_sim = jnp.take(vals[:, :K_STATIC], cols, axis=1)
    retrieved_ids = jnp.take(ids[:, :K_STATIC], cols, axis=1)
    embeds = _make_sc_gather(D, Q * K_STATIC, 640, False)(
        index, retrieved_ids.reshape(-1))
    retrieved_embeds = embeds.reshape(Q, K_STATIC, D)
    return top_sim, retrieved_ids, retrieved_embeds
